# baseline (speedup 1.0000x reference)
import functools

import jax
import jax.numpy as jnp
from jax import lax
from jax.experimental import pallas as pl
from jax.experimental.pallas import tpu as pltpu
from jax.experimental.pallas import tpu_sc as plsc

_NC = 2
_NS = 16
_NW = _NC * _NS

_BATCH = 16384
_N_FIELDS = 26
_DIM = 32
_TOTAL = _BATCH * _N_FIELDS
_BLK = 128
_NBLK = _TOTAL // _BLK
_BLK_PER_W = _NBLK // _NW
_IDX_PER_W = _BLK_PER_W * _BLK
_CBLK = _BATCH // _BLK


def _make_gather():
    mesh = plsc.VectorSubcoreMesh(core_axis_name="c", subcore_axis_name="s")

    @functools.partial(
        pl.kernel,
        mesh=mesh,
        compiler_params=pltpu.CompilerParams(use_tc_tiling_on_sc=False,
                                             needs_layout_passes=False),
        out_type=jax.ShapeDtypeStruct((_N_FIELDS, 4, _CBLK, 8, 128),
                                      jnp.float32),
        scratch_types=[
            pltpu.VMEM((_IDX_PER_W,), jnp.int32),
            pltpu.VMEM((_BLK, _DIM), jnp.float32),
            pltpu.VMEM((_BLK, _DIM), jnp.float32),
            pltpu.VMEM((4, 8, 128), jnp.float32),
            pltpu.VMEM((4, 8, 128), jnp.float32),
            pltpu.SemaphoreType.DMA,
            pltpu.SemaphoreType.DMA,
            pltpu.SemaphoreType.DMA,
            pltpu.SemaphoreType.DMA,
        ],
    )
    def gather_kernel(idx_hbm, table_hbm, out_hbm,
                      idx_v, rows0, rows1, tiles0, tiles1, g0, g1, w0, w1):
        wid = lax.axis_index("s") * _NC + lax.axis_index("c")
        base_blk = wid * _BLK_PER_W
        pltpu.sync_copy(idx_hbm.at[pl.ds(wid * _IDX_PER_W, _IDX_PER_W)],
                        idx_v)

        rows = (rows0, rows1)
        tiles = (tiles0, tiles1)
        gsem = (g0, g1)
        wsem = (w0, w1)
        lane = lax.broadcasted_iota(jnp.int32, (16,), 0)

        def gather_copy(jblk, b):
            return pltpu.make_async_copy(
                table_hbm.at[idx_v.at[pl.ds(jblk * _BLK, _BLK)]],
                rows[b], gsem[b])

        def write_copy(g, b, r):
            f = g // _CBLK
            c = lax.rem(g, _CBLK)
            return pltpu.make_async_copy(
                tiles[b].at[r], out_hbm.at[f, r, c], wsem[b])

        for b in (0, 1):
            gather_copy(b, b).start()

        def body(i, carry):
            for b in (0, 1):
                jblk = 2 * i + b
                g = base_blk + jblk

                @pl.when(i >= 1)
                def _wait_writes():
                    for r in range(4):
                        write_copy(g, b, r).wait()

                gather_copy(jblk, b).wait()

                for r in range(4):
                    for s in range(8):
                        col = jnp.full((16,), 8 * r + s, jnp.int32)
                        for e in range(8):
                            vec = plsc.load_gather(
                                rows[b], [lane + 16 * e, col])
                            tiles[b][r, s, pl.ds(16 * e, 16)] = vec

                for r in range(4):
                    write_copy(g, b, r).start()

                @pl.when(i < (_BLK_PER_W // 2) - 1)
                def _next_gather():
                    gather_copy(jblk + 2, b).start()
            return carry

        lax.fori_loop(0, _BLK_PER_W // 2, body, 0)

        for b in (0, 1):
            g = base_blk + _BLK_PER_W - 2 + b
            for r in range(4):
                write_copy(g, b, r).wait()

    return gather_kernel


_gather = _make_gather()


def kernel(x, table):
    idx = x.T.reshape(_TOTAL)
    m = _gather(idx, table)
    return m.transpose(2, 4, 0, 1, 3).reshape(_BATCH, _N_FIELDS, _DIM)


# speedup vs baseline: 1.1816x; 1.1816x over previous
import functools

import jax
import jax.numpy as jnp
from jax import lax
from jax.experimental import pallas as pl
from jax.experimental.pallas import tpu as pltpu
from jax.experimental.pallas import tpu_sc as plsc

_NC = 2
_NS = 16
_NW = _NC * _NS

_BATCH = 16384
_N_FIELDS = 26
_DIM = 32
_TOTAL = _BATCH * _N_FIELDS
_BLK = 128
_NBLK = _TOTAL // _BLK
_BLK_PER_W = _NBLK // _NW
_IDX_PER_W = _BLK_PER_W * _BLK
_CBLK = _BATCH // _BLK


def _make_gather():
    mesh = plsc.VectorSubcoreMesh(core_axis_name="c", subcore_axis_name="s")

    @functools.partial(
        pl.kernel,
        mesh=mesh,
        compiler_params=pltpu.CompilerParams(use_tc_tiling_on_sc=False,
                                             needs_layout_passes=False),
        out_type=jax.ShapeDtypeStruct((_N_FIELDS, 4, _CBLK, 8, 128),
                                      jnp.float32),
        scratch_types=[
            pltpu.VMEM((_IDX_PER_W,), jnp.int32),
            pltpu.VMEM((_BLK, _DIM), jnp.float32),
            pltpu.VMEM((_BLK, _DIM), jnp.float32),
            pltpu.VMEM((4, 8, 128), jnp.float32),
            pltpu.VMEM((4, 8, 128), jnp.float32),
            pltpu.SemaphoreType.DMA,
            pltpu.SemaphoreType.DMA,
            pltpu.SemaphoreType.DMA,
            pltpu.SemaphoreType.DMA,
        ],
    )
    def gather_kernel(idx_hbm, table_hbm, out_hbm,
                      idx_v, rows0, rows1, tiles0, tiles1, g0, g1, w0, w1):
        wid = lax.axis_index("s") * _NC + lax.axis_index("c")
        base_blk = wid * _BLK_PER_W
        pltpu.sync_copy(idx_hbm.at[pl.ds(wid * _IDX_PER_W, _IDX_PER_W)],
                        idx_v)

        rows = (rows0, rows1)
        tiles = (tiles0, tiles1)
        gsem = (g0, g1)
        wsem = (w0, w1)
        lane = lax.broadcasted_iota(jnp.int32, (16,), 0)

        def gather_copy(jblk, b):
            return pltpu.make_async_copy(
                table_hbm.at[idx_v.at[pl.ds(jblk * _BLK, _BLK)]],
                rows[b], gsem[b])

        def write_copy(g, b, r):
            f = g // _CBLK
            c = lax.rem(g, _CBLK)
            return pltpu.make_async_copy(
                tiles[b].at[r], out_hbm.at[f, r, c], wsem[b])

        for b in (0, 1):
            gather_copy(b, b).start()

        def body(i, carry):
            for b in (0, 1):
                jblk = 2 * i + b
                g = base_blk + jblk

                @pl.when(i >= 1)
                def _wait_writes():
                    for r in range(4):
                        write_copy(g, b, r).wait()

                gather_copy(jblk, b).wait()

                rows_b = rows[b]
                tiles_b = tiles[b]

                @plsc.parallel_loop(0, 256, unroll=8)
                def _transpose(it):
                    j = it // 8
                    e = lax.rem(it, 8)
                    col = jnp.full((16,), j, jnp.int32)
                    vec = plsc.load_gather(rows_b, [lane + 16 * e, col])
                    tiles_b[j // 8, lax.rem(j, 8), pl.ds(16 * e, 16)] = vec

                for r in range(4):
                    write_copy(g, b, r).start()

                @pl.when(i < (_BLK_PER_W // 2) - 1)
                def _next_gather():
                    gather_copy(jblk + 2, b).start()
            return carry

        lax.fori_loop(0, _BLK_PER_W // 2, body, 0)

        for b in (0, 1):
            g = base_blk + _BLK_PER_W - 2 + b
            for r in range(4):
                write_copy(g, b, r).wait()

    return gather_kernel


_gather = _make_gather()


def kernel(x, table):
    idx = x.T.reshape(_TOTAL)
    m = _gather(idx, table)
    return m.transpose(2, 4, 0, 1, 3).reshape(_BATCH, _N_FIELDS, _DIM)
